# baseline (speedup 1.0000x reference)
import functools

import jax
import jax.numpy as jnp
from jax import lax
from jax.experimental import pallas as pl
from jax.experimental.pallas import tpu as pltpu
from jax.experimental.pallas import tpu_sc as plsc

VOCAB = 100000
EMBED = 128
BATCH = 16384
HIST = 200

NUM_WORKERS = 32
ROWS_PER_WORKER = BATCH // NUM_WORKERS
HALF = HIST // 2


G = 64
NSLAB = ROWS_PER_WORKER // G
KPAD = 104


def _sc_gather_pool(table, ids_pad):
  mesh = plsc.VectorSubcoreMesh(core_axis_name="c", subcore_axis_name="s")

  @functools.partial(
      pl.kernel,
      out_type=jax.ShapeDtypeStruct((BATCH, EMBED), jnp.bfloat16),
      mesh=mesh,
      compiler_params=pltpu.CompilerParams(use_tc_tiling_on_sc=False),
      scratch_types=[
          pltpu.VMEM((2 * G, KPAD), jnp.int32),
          pltpu.VMEM((2 * G, KPAD), jnp.int32),
          pltpu.VMEM((2 * G, KPAD), jnp.int32),
          pltpu.VMEM((KPAD, EMBED), jnp.bfloat16),
          pltpu.VMEM((KPAD, EMBED), jnp.bfloat16),
          pltpu.VMEM((KPAD, EMBED), jnp.bfloat16),
          pltpu.VMEM((KPAD, EMBED), jnp.bfloat16),
          pltpu.VMEM((ROWS_PER_WORKER, EMBED), jnp.bfloat16),
          pltpu.SemaphoreType.DMA,
          pltpu.SemaphoreType.DMA,
          pltpu.SemaphoreType.DMA,
      ],
  )
  def k(table_hbm, ids_hbm, out_hbm, idx0, idx1, idx2, buf_a0, buf_a1,
        buf_b0, buf_b1, stage, sem_a, sem_b, sem_i):
    idxs = (idx0, idx1, idx2)
    wid = lax.axis_index("s") * 2 + lax.axis_index("c")
    base = wid * ROWS_PER_WORKER

    def slab_src(kk):
      return ids_hbm.at[pl.ds((base + kk * G) * 2, 2 * G)]

    def start(idx_v, j, buf0, buf1, sem):
      pltpu.async_copy(table_hbm.at[idx_v.at[0]], buf0, sem)
      pltpu.async_copy(table_hbm.at[idx_v.at[1]], buf1, sem)

    def wait_slot(buf0, buf1, sem):
      pltpu.make_async_copy(table_hbm.at[pl.ds(0, KPAD)], buf0, sem).wait()
      pltpu.make_async_copy(table_hbm.at[pl.ds(0, KPAD)], buf1, sem).wait()

    def reduce_store(rel, buf0, buf1):
      def body0(r, accs):
        return tuple(accs[q] + buf0[r, pl.ds(q * 32, 32)] for q in range(4))

      def body1(r, accs):
        return tuple(accs[q] + buf1[r, pl.ds(q * 32, 32)] for q in range(4))

      accs = tuple(jnp.zeros((32,), jnp.bfloat16) for _ in range(4))
      for q in range(4):
        stage[rel, pl.ds(q * 32, 32)] = accs[q]

    pltpu.sync_copy(slab_src(0), idx0)
    pltpu.sync_copy(slab_src(1), idx1)
    start(idx0, 0, buf_a0, buf_a1, sem_a)

    for kk in range(NSLAB):
      idx_cur = idxs[kk % 3]
      idx_next = idxs[(kk + 1) % 3]
      if 1 <= kk <= NSLAB - 2:
        pltpu.make_async_copy(slab_src(0), idx_next, sem_i).wait()
      if kk <= NSLAB - 3:
        pltpu.async_copy(slab_src(kk + 2), idxs[(kk + 2) % 3], sem_i)

      @pl.loop(0, G // 2)
      def _(i, kk=kk, idx_cur=idx_cur, idx_next=idx_next):
        rel0 = kk * G + 2 * i
        start(idx_cur, 2 * i + 1, buf_b0, buf_b1, sem_b)
        wait_slot(buf_a0, buf_a1, sem_a)
        reduce_store(rel0, buf_a0, buf_a1)

        @pl.when(i < G // 2 - 1)
        def _():
          start(idx_cur, 2 * i + 2, buf_a0, buf_a1, sem_a)

        if kk < NSLAB - 1:
          @pl.when(i == G // 2 - 1)
          def _():
            start(idx_next, 0, buf_a0, buf_a1, sem_a)

        wait_slot(buf_b0, buf_b1, sem_b)
        reduce_store(rel0 + 1, buf_b0, buf_b1)

    pltpu.sync_copy(stage, out_hbm.at[pl.ds(base, ROWS_PER_WORKER)])

  return k(table, ids_pad)


def _erf(x):
  ax = jnp.abs(x)
  t = 1.0 / (1.0 + 0.3275911 * ax)
  poly = t * (0.254829592 + t * (-0.284496736 + t * (1.421413741 + t * (
      -1.453152027 + t * 1.061405429))))
  y = 1.0 - poly * jnp.exp(-ax * ax)
  return jnp.sign(x) * y


def _gelu_exact(x):
  return 0.5 * x * (1.0 + _erf(x * 0.7071067811865476))


def _tc_mlp(pooled, age, age_w1, age_b1, age_w2, age_b2, fin_w1, fin_b1,
            fin_w2r, fin_b2):
  tile = 2048
  grid = BATCH // tile

  def body(pooled_ref, age_ref, w1_ref, b1_ref, w2_ref, b2_ref, fw1_ref,
           fb1_ref, fw2_ref, fb2_ref, out_ref):
    hob = pooled_ref[...].astype(jnp.float32) * (1.0 / HIST)
    a_pre = age_ref[...] * w1_ref[...] + b1_ref[...]
    a = _gelu_exact(a_pre)
    age_e = jnp.dot(a, w2_ref[...], preferred_element_type=jnp.float32) + b2_ref[...]
    comb = jnp.concatenate([hob, age_e], axis=1)
    h_pre = jnp.dot(comb, fw1_ref[...], preferred_element_type=jnp.float32) + fb1_ref[...]
    h = _gelu_exact(h_pre)
    out_ref[...] = (
        jnp.sum(h * fw2_ref[...], axis=1, keepdims=True) + fb2_ref[...]
    )

  full = lambda shape: pl.BlockSpec(shape, lambda i: (0, 0))
  return pl.pallas_call(
      body,
      grid=(grid,),
      in_specs=[
          pl.BlockSpec((tile, EMBED), lambda i: (i, 0)),
          pl.BlockSpec((tile, 1), lambda i: (i, 0)),
          full((1, 16)),
          full((1, 16)),
          full((16, EMBED)),
          full((1, EMBED)),
          full((2 * EMBED, 64)),
          full((1, 64)),
          full((1, 64)),
          full((1, 1)),
      ],
      out_specs=pl.BlockSpec((tile, 1), lambda i: (i, 0)),
      out_shape=jax.ShapeDtypeStruct((BATCH, 1), jnp.float32),
  )(pooled, age, age_w1, age_b1, age_w2, age_b2, fin_w1, fin_b1, fin_w2r,
    fin_b2)


def kernel(hobbies_ids, age_tensor, emb_table, age_w1, age_b1, age_w2, age_b2,
           fin_w1, fin_b1, fin_w2, fin_b2):
  ids2 = hobbies_ids.astype(jnp.int32).reshape(2 * BATCH, HALF)
  ids_pad = jnp.pad(ids2, ((0, 0), (0, KPAD - HALF)))
  pooled = _sc_gather_pool(emb_table.astype(jnp.bfloat16), ids_pad)
  return _tc_mlp(
      pooled,
      age_tensor,
      age_w1,
      age_b1.reshape(1, 16),
      age_w2,
      age_b2.reshape(1, EMBED),
      fin_w1,
      fin_b1.reshape(1, 64),
      fin_w2.reshape(1, 64),
      fin_b2.reshape(1, 1),
  )


# speedup vs baseline: 4.0837x; 4.0837x over previous
import functools

import jax
import jax.numpy as jnp
from jax import lax
from jax.experimental import pallas as pl
from jax.experimental.pallas import tpu as pltpu
from jax.experimental.pallas import tpu_sc as plsc

VOCAB = 100000
EMBED = 128
BATCH = 16384
HIST = 200

NUM_WORKERS = 32
ROWS_PER_WORKER = BATCH // NUM_WORKERS
HALF = HIST // 2


G = 64
NSLAB = ROWS_PER_WORKER // G
KPAD = 104


def _sc_gather_pool(table, ids_pad):
  mesh = plsc.VectorSubcoreMesh(core_axis_name="c", subcore_axis_name="s")

  @functools.partial(
      pl.kernel,
      out_type=jax.ShapeDtypeStruct((BATCH, EMBED), jnp.bfloat16),
      mesh=mesh,
      compiler_params=pltpu.CompilerParams(use_tc_tiling_on_sc=False),
      scratch_types=[
          pltpu.VMEM((2 * G, KPAD), jnp.int32),
          pltpu.VMEM((2 * G, KPAD), jnp.int32),
          pltpu.VMEM((2 * G, KPAD), jnp.int32),
          pltpu.VMEM((KPAD, EMBED), jnp.bfloat16),
          pltpu.VMEM((KPAD, EMBED), jnp.bfloat16),
          pltpu.VMEM((KPAD, EMBED), jnp.bfloat16),
          pltpu.VMEM((KPAD, EMBED), jnp.bfloat16),
          pltpu.VMEM((ROWS_PER_WORKER, EMBED), jnp.bfloat16),
          pltpu.SemaphoreType.DMA,
          pltpu.SemaphoreType.DMA,
          pltpu.SemaphoreType.DMA,
      ],
  )
  def k(table_hbm, ids_hbm, out_hbm, idx0, idx1, idx2, buf_a0, buf_a1,
        buf_b0, buf_b1, stage, sem_a, sem_b, sem_i):
    idxs = (idx0, idx1, idx2)
    wid = lax.axis_index("s") * 2 + lax.axis_index("c")
    base = wid * ROWS_PER_WORKER

    def slab_src(kk):
      return ids_hbm.at[pl.ds((base + kk * G) * 2, 2 * G)]

    def start(idx_v, j, buf0, buf1, sem):
      pltpu.async_copy(table_hbm.at[idx_v.at[2 * j]], buf0, sem)
      pltpu.async_copy(table_hbm.at[idx_v.at[2 * j + 1]], buf1, sem)

    def wait_slot(buf0, buf1, sem):
      pltpu.make_async_copy(table_hbm.at[pl.ds(0, KPAD)], buf0, sem).wait()
      pltpu.make_async_copy(table_hbm.at[pl.ds(0, KPAD)], buf1, sem).wait()

    def reduce_store(rel, buf0, buf1):
      def body0(r, accs):
        return tuple(accs[q] + buf0[r, pl.ds(q * 32, 32)] for q in range(4))

      def body1(r, accs):
        return tuple(accs[q] + buf1[r, pl.ds(q * 32, 32)] for q in range(4))

      accs = tuple(jnp.zeros((32,), jnp.bfloat16) for _ in range(4))
      accs = lax.fori_loop(0, HALF, body0, accs)
      accs = lax.fori_loop(0, HALF, body1, accs)
      for q in range(4):
        stage[rel, pl.ds(q * 32, 32)] = accs[q]

    pltpu.sync_copy(slab_src(0), idx0)
    pltpu.sync_copy(slab_src(1), idx1)
    start(idx0, 0, buf_a0, buf_a1, sem_a)

    for kk in range(NSLAB):
      idx_cur = idxs[kk % 3]
      idx_next = idxs[(kk + 1) % 3]
      if 1 <= kk <= NSLAB - 2:
        pltpu.make_async_copy(slab_src(0), idx_next, sem_i).wait()
      if kk <= NSLAB - 3:
        pltpu.async_copy(slab_src(kk + 2), idxs[(kk + 2) % 3], sem_i)

      @pl.loop(0, G // 2)
      def _(i, kk=kk, idx_cur=idx_cur, idx_next=idx_next):
        rel0 = kk * G + 2 * i
        start(idx_cur, 2 * i + 1, buf_b0, buf_b1, sem_b)
        wait_slot(buf_a0, buf_a1, sem_a)
        reduce_store(rel0, buf_a0, buf_a1)

        @pl.when(i < G // 2 - 1)
        def _():
          start(idx_cur, 2 * i + 2, buf_a0, buf_a1, sem_a)

        if kk < NSLAB - 1:
          @pl.when(i == G // 2 - 1)
          def _():
            start(idx_next, 0, buf_a0, buf_a1, sem_a)

        wait_slot(buf_b0, buf_b1, sem_b)
        reduce_store(rel0 + 1, buf_b0, buf_b1)

    pltpu.sync_copy(stage, out_hbm.at[pl.ds(base, ROWS_PER_WORKER)])

  return k(table, ids_pad)


def _erf(x):
  ax = jnp.abs(x)
  t = 1.0 / (1.0 + 0.3275911 * ax)
  poly = t * (0.254829592 + t * (-0.284496736 + t * (1.421413741 + t * (
      -1.453152027 + t * 1.061405429))))
  y = 1.0 - poly * jnp.exp(-ax * ax)
  return jnp.sign(x) * y


def _gelu_exact(x):
  return 0.5 * x * (1.0 + _erf(x * 0.7071067811865476))


def _tc_mlp(pooled, age, age_w1, age_b1, age_w2, age_b2, fin_w1, fin_b1,
            fin_w2r, fin_b2):
  tile = 2048
  grid = BATCH // tile

  def body(pooled_ref, age_ref, w1_ref, b1_ref, w2_ref, b2_ref, fw1_ref,
           fb1_ref, fw2_ref, fb2_ref, out_ref):
    hob = pooled_ref[...].astype(jnp.float32) * (1.0 / HIST)
    a_pre = age_ref[...] * w1_ref[...] + b1_ref[...]
    a = _gelu_exact(a_pre)
    age_e = jnp.dot(a, w2_ref[...], preferred_element_type=jnp.float32) + b2_ref[...]
    comb = jnp.concatenate([hob, age_e], axis=1)
    h_pre = jnp.dot(comb, fw1_ref[...], preferred_element_type=jnp.float32) + fb1_ref[...]
    h = _gelu_exact(h_pre)
    out_ref[...] = (
        jnp.sum(h * fw2_ref[...], axis=1, keepdims=True) + fb2_ref[...]
    )

  full = lambda shape: pl.BlockSpec(shape, lambda i: (0, 0))
  return pl.pallas_call(
      body,
      grid=(grid,),
      in_specs=[
          pl.BlockSpec((tile, EMBED), lambda i: (i, 0)),
          pl.BlockSpec((tile, 1), lambda i: (i, 0)),
          full((1, 16)),
          full((1, 16)),
          full((16, EMBED)),
          full((1, EMBED)),
          full((2 * EMBED, 64)),
          full((1, 64)),
          full((1, 64)),
          full((1, 1)),
      ],
      out_specs=pl.BlockSpec((tile, 1), lambda i: (i, 0)),
      out_shape=jax.ShapeDtypeStruct((BATCH, 1), jnp.float32),
  )(pooled, age, age_w1, age_b1, age_w2, age_b2, fin_w1, fin_b1, fin_w2r,
    fin_b2)


def kernel(hobbies_ids, age_tensor, emb_table, age_w1, age_b1, age_w2, age_b2,
           fin_w1, fin_b1, fin_w2, fin_b2):
  ids2 = hobbies_ids.astype(jnp.int32).reshape(2 * BATCH, HALF)
  ids_pad = jnp.pad(ids2, ((0, 0), (0, KPAD - HALF)), mode='edge')
  pooled = _sc_gather_pool(emb_table.astype(jnp.bfloat16), ids_pad)
  return _tc_mlp(
      pooled,
      age_tensor,
      age_w1,
      age_b1.reshape(1, 16),
      age_w2,
      age_b2.reshape(1, EMBED),
      fin_w1,
      fin_b1.reshape(1, 64),
      fin_w2.reshape(1, 64),
      fin_b2.reshape(1, 1),
  )
